# Initial kernel scaffold; baseline (speedup 1.0000x reference)
import jax
import jax.numpy as jnp
from jax.experimental import pallas as pl


def kernel(input_ids, word_table, pos_table, gamma, beta):
    raise NotImplementedError("write your pallas kernel here")


# speedup vs baseline: 2.6589x; 2.6589x over previous
import functools

import jax
import jax.numpy as jnp
from jax import lax
from jax.experimental import pallas as pl
from jax.experimental.pallas import tpu as pltpu
from jax.experimental.pallas import tpu_sc as plsc

VOCAB = 100000
HID = 128
MAXPOS = 512
B = 1024
S = 200
EPS = 1e-12

NC = 2
NS = 16
NW = NC * NS
ROWS_PER_W = B * S // NW
CH = 128
NCHUNK = ROWS_PER_W // CH
NSEG = HID // 16

_mesh = plsc.VectorSubcoreMesh(core_axis_name="c", subcore_axis_name="s")


@functools.partial(
    pl.kernel,
    out_type=jax.ShapeDtypeStruct((B * S, HID), jnp.float32),
    mesh=_mesh,
    scratch_types=[
        pltpu.VMEM((NCHUNK, CH), jnp.int32),
        pltpu.VMEM((S, HID), jnp.float32),
        pltpu.VMEM((HID,), jnp.float32),
        pltpu.VMEM((HID,), jnp.float32),
        pltpu.VMEM((2, CH, HID), jnp.float32),
        pltpu.VMEM((2, CH, HID), jnp.float32),
        pltpu.SemaphoreType.DMA,
        pltpu.SemaphoreType.DMA,
        pltpu.SemaphoreType.DMA,
        pltpu.SemaphoreType.DMA,
    ],
)
def _embed_ln(ids_hbm, wt_hbm, pos_hbm, g_hbm, b_hbm, out_hbm,
              ids_v, pos_v, g_v, b_v, ibuf, obuf, gs0, gs1, ws0, ws1):
    wid = lax.axis_index("s") * NC + lax.axis_index("c")
    wbase = wid * ROWS_PER_W
    gsem = (gs0, gs1)
    wsem = (ws0, ws1)

    pltpu.sync_copy(ids_hbm.at[wid], ids_v)

    def fire_gather(c, j):
        pltpu.make_async_copy(
            wt_hbm.at[ids_v.at[c]], ibuf.at[j], gsem[j]).start()

    def wait_gather(c, j):
        pltpu.make_async_copy(
            wt_hbm.at[ids_v.at[c]], ibuf.at[j], gsem[j]).wait()

    def fire_write(c, j):
        pltpu.make_async_copy(
            obuf.at[j], out_hbm.at[pl.ds(wbase + c * CH, CH)],
            wsem[j]).start()

    def wait_write(c, j):
        pltpu.make_async_copy(
            obuf.at[j], out_hbm.at[pl.ds(wbase + c * CH, CH)],
            wsem[j]).wait()

    fire_gather(0, 0)
    fire_gather(1, 1)

    pltpu.sync_copy(pos_hbm.at[pl.ds(0, S)], pos_v)
    pltpu.sync_copy(g_hbm, g_v)
    pltpu.sync_copy(b_hbm, b_v)

    gamma = [g_v[pl.ds(16 * k, 16)] for k in range(NSEG)]
    beta = [b_v[pl.ds(16 * k, 16)] for k in range(NSEG)]
    lanes = lax.iota(jnp.int32, 16)

    def hsum(v):
        for sh in (8, 4, 2, 1):
            v = v + v.at[lanes ^ sh].get(mode="promise_in_bounds")
        return v

    def ln_chunk(c, j):
        inb = ibuf.at[j]
        outb = obuf.at[j]
        base = c * CH

        def row(r, carry):
            prow = lax.rem(base + r, S)
            xs = []
            for k in range(NSEG):
                w = inb[r, pl.ds(16 * k, 16)]
                p = pos_v[prow, pl.ds(16 * k, 16)]
                xs.append(w + p)
            s = xs[0]
            q = xs[0] * xs[0]
            for k in range(1, NSEG):
                s = s + xs[k]
                q = q + xs[k] * xs[k]
            mv = hsum(s) * (1.0 / HID)
            vv = hsum(q) * (1.0 / HID) - mv * mv + EPS
            ii = lax.bitcast_convert_type(vv, jnp.int32)
            y = lax.bitcast_convert_type(
                jnp.int32(0x5F3759DF) - (ii >> 1), jnp.float32)
            for _ in range(3):
                y = y * (1.5 - 0.5 * vv * y * y)
            for k in range(NSEG):
                a = gamma[k] * y
                outb[r, pl.ds(16 * k, 16)] = (xs[k] - mv) * a + beta[k]
            return carry

        lax.fori_loop(0, CH, row, 0, unroll=2)

    def chunk(c, j, first, last):
        wait_gather(c, j)
        if not first:
            wait_write(c, j)
        ln_chunk(c, j)
        fire_write(c, j)
        if not last:
            fire_gather(c + 2, j)

    chunk(0, 0, True, False)
    chunk(1, 1, True, False)

    def pair(i, carry):
        chunk(2 * i, 0, False, False)
        chunk(2 * i + 1, 1, False, False)
        return carry

    lax.fori_loop(1, NCHUNK // 2 - 1, pair, 0)

    chunk(NCHUNK - 2, 0, False, True)
    chunk(NCHUNK - 1, 1, False, True)

    wait_write(NCHUNK - 2, 0)
    wait_write(NCHUNK - 1, 1)


def kernel(input_ids, word_table, pos_table, gamma, beta):
    ids = input_ids.astype(jnp.int32).reshape(NW, NCHUNK, CH)
    out = _embed_ln(ids, word_table, pos_table, gamma, beta)
    return out.reshape(B, S, HID)
